# Initial kernel scaffold; baseline (speedup 1.0000x reference)
import jax
import jax.numpy as jnp
from jax import lax
from jax.experimental import pallas as pl
from jax.experimental.pallas import tpu as pltpu
from jax.experimental.pallas import tpu_sc as plsc

N = 650
E = 41600
HID = 256
BS = 64
NC = 2
NS = 16
NW = NC * NS
GRP = 128
G = 11
EPW = G * GRP
E_PAD = NW * EPW
TBL = N * N
TBL_PAD = 422912
CHW = TBL_PAD // NS


def _sc_count_body(src_hbm, dst_hbm, out_hbm, src_v, dst_v, idx_v, ones_v,
                   zbuf, tbl_sh):
    c = lax.axis_index("c")
    s = lax.axis_index("s")
    wid = s * NC + c

    zero16 = jnp.zeros((16,), jnp.float32)

    def _zf(i, carry):
        zbuf[pl.ds(i * 16, 16)] = zero16
        return carry

    lax.fori_loop(0, CHW // 16, _zf, 0)

    one16 = jnp.ones((16,), jnp.float32)
    for j in range(GRP // 16):
        ones_v[pl.ds(j * 16, 16)] = one16

    pltpu.sync_copy(zbuf, tbl_sh.at[pl.ds(s * CHW, CHW)])

    base = wid * EPW
    pltpu.sync_copy(src_hbm.at[pl.ds(base, EPW)], src_v)
    pltpu.sync_copy(dst_hbm.at[pl.ds(base, EPW)], dst_v)

    for g in range(G):
        for j in range(GRP // 16):
            o = g * GRP + j * 16
            idx_v[g, pl.ds(j * 16, 16)] = (
                dst_v[pl.ds(o, 16)] * N + src_v[pl.ds(o, 16)])

    plsc.subcore_barrier()
    for g in range(G):
        pltpu.sync_copy(ones_v, tbl_sh.at[idx_v.at[g]], add=True)
    plsc.subcore_barrier()

    off = s * CHW
    pltpu.sync_copy(tbl_sh.at[pl.ds(off, CHW)],
                    out_hbm.at[pl.ds(c * TBL_PAD + off, CHW)])


_sc_count = pl.kernel(
    _sc_count_body,
    out_type=jax.ShapeDtypeStruct((NC * TBL_PAD,), jnp.float32),
    mesh=plsc.VectorSubcoreMesh(core_axis_name="c", subcore_axis_name="s",
                                num_cores=NC, num_subcores=NS),
    scratch_types=[
        pltpu.VMEM((EPW,), jnp.int32),
        pltpu.VMEM((EPW,), jnp.int32),
        pltpu.VMEM((G, GRP), jnp.int32),
        pltpu.VMEM((GRP,), jnp.float32),
        pltpu.VMEM((CHW,), jnp.float32),
        pltpu.VMEM_SHARED((TBL_PAD,), jnp.float32),
    ],
)


def _dense_body(x_ref, c0_ref, c1_ref, w1_ref, as1_ref, ad1_ref, b1_ref,
                w4_ref, as4_ref, ad4_ref, b4_ref, out_ref):
    f32 = jnp.float32
    C = c0_ref[:] + c1_ref[:]
    rows = lax.broadcasted_iota(jnp.int32, (N, N), 0)
    cols = lax.broadcasted_iota(jnp.int32, (N, N), 1)
    C = C + jnp.where(rows == cols, 1.0, 0.0).astype(f32)
    mask = C > 0.0

    def gat(h, att_s, att_d, b):
        a_s = lax.dot_general(att_s, h, (((1,), (1,)), ((), ())),
                              preferred_element_type=f32)
        a_d = jnp.sum(h * att_d, axis=1, keepdims=True)
        alpha = a_d + a_s
        alpha = jnp.where(alpha > 0.0, alpha, 0.2 * alpha)
        am = jnp.max(jnp.where(mask, alpha, -1e30), axis=1, keepdims=True)
        e = jnp.where(mask, C * jnp.exp(alpha - am), 0.0)
        denom = jnp.sum(e, axis=1, keepdims=True)
        coef = e / denom
        return jnp.dot(coef, h, preferred_element_type=f32) + b

    h1 = jnp.dot(x_ref[:], w1_ref[:], preferred_element_type=f32)
    h = jnp.maximum(gat(h1, as1_ref[:], ad1_ref[:], b1_ref[:]), 0.0)
    h2 = jnp.dot(h, w4_ref[:], preferred_element_type=f32)
    g = gat(h2, as4_ref[:], ad4_ref[:], b4_ref[:])
    out_ref[:] = jnp.where(g > 0.0, g, 0.01 * g)


def _fc_body(hr_ref, wfc_ref, bfc_ref, out_ref):
    o = jnp.dot(hr_ref[:], wfc_ref[:],
                preferred_element_type=jnp.float32) + bfc_ref[:]
    out_ref[:] = 1.0 / (1.0 + jnp.exp(-o))


def kernel(x_s, x_t, edge_index, edge_attr, batch, W1, att_src1, att_dst1,
           b1, W4, att_src4, att_dst4, b4, Wfc, bfc):
    x = jnp.concatenate([x_s, x_t], axis=0)
    x = jnp.pad(x, ((0, N - x.shape[0]), (0, 0)))

    padn = E_PAD - E
    src = jnp.concatenate(
        [edge_index[0], jnp.zeros((padn,), edge_index.dtype)])
    dst = jnp.concatenate(
        [edge_index[1], jnp.full((padn,), N, edge_index.dtype)])

    tbl = _sc_count(src, dst)
    c0 = tbl[:TBL].reshape(N, N)
    c1 = tbl[TBL_PAD:TBL_PAD + TBL].reshape(N, N)

    g = pl.pallas_call(
        _dense_body,
        out_shape=jax.ShapeDtypeStruct((N, BS), jnp.float32),
    )(x, c0, c1, W1,
      att_src1.reshape(1, HID), att_dst1.reshape(1, HID), b1.reshape(1, HID),
      W4, att_src4.reshape(1, BS), att_dst4.reshape(1, BS), b4.reshape(1, BS))

    hr = g.reshape(BS, N)
    out = pl.pallas_call(
        _fc_body,
        out_shape=jax.ShapeDtypeStruct((BS, 1), jnp.float32),
    )(hr, Wfc, bfc.reshape(1, 1))
    return out.reshape(1, BS)


# speedup vs baseline: 56.5708x; 56.5708x over previous
import jax
import jax.numpy as jnp
from jax import lax
from jax.experimental import pallas as pl
from jax.experimental.pallas import tpu as pltpu
from jax.experimental.pallas import tpu_sc as plsc

N = 650
E = 41600
HID = 256
BS = 64
NC = 2
NS = 16
NW = NC * NS
GRP = 128
G = 11
EPW = G * GRP
E_PAD = NW * EPW
TBL = N * N
TBL_PAD = 422912
CHW = TBL_PAD // NS


def _sc_count_body(src_hbm, dst_hbm, out_hbm, src_v, dst_v, idx_v, ones_v,
                   zbuf, tbl_sh):
    c = lax.axis_index("c")
    s = lax.axis_index("s")
    wid = s * NC + c

    zero16 = jnp.zeros((16,), jnp.float32)

    def _zf(i, carry):
        zbuf[pl.ds(i * 16, 16)] = zero16
        return carry

    lax.fori_loop(0, CHW // 16, _zf, 0)

    one16 = jnp.ones((16,), jnp.float32)
    for j in range(GRP // 16):
        ones_v[pl.ds(j * 16, 16)] = one16

    pltpu.sync_copy(zbuf, tbl_sh.at[pl.ds(s * CHW, CHW)])

    base = wid * EPW
    pltpu.sync_copy(src_hbm.at[pl.ds(base, EPW)], src_v)
    pltpu.sync_copy(dst_hbm.at[pl.ds(base, EPW)], dst_v)

    for g in range(G):
        for j in range(GRP // 16):
            o = g * GRP + j * 16
            idx_v[g, pl.ds(j * 16, 16)] = (
                dst_v[pl.ds(o, 16)] * N + src_v[pl.ds(o, 16)])

    plsc.subcore_barrier()
    for g in range(G):
        pltpu.sync_copy(ones_v, tbl_sh.at[idx_v.at[g]], add=True)
    plsc.subcore_barrier()

    off = s * CHW
    pltpu.sync_copy(tbl_sh.at[pl.ds(off, CHW)], zbuf)
    pltpu.sync_copy(zbuf, out_hbm.at[pl.ds(c * TBL_PAD + off, CHW)])


_SC_COUNT_CACHE = []


def _sc_count(src, dst):
    if not _SC_COUNT_CACHE:
        _SC_COUNT_CACHE.append(pl.kernel(
            _sc_count_body,
            out_type=jax.ShapeDtypeStruct((NC * TBL_PAD,), jnp.float32),
            mesh=plsc.VectorSubcoreMesh(core_axis_name="c",
                                        subcore_axis_name="s",
                                        num_cores=NC, num_subcores=NS),
            scratch_types=[
                pltpu.VMEM((EPW,), jnp.int32),
                pltpu.VMEM((EPW,), jnp.int32),
                pltpu.VMEM((G, GRP), jnp.int32),
                pltpu.VMEM((GRP,), jnp.float32),
                pltpu.VMEM((CHW,), jnp.float32),
                pltpu.VMEM_SHARED((TBL_PAD,), jnp.float32),
            ],
        ))
    return _SC_COUNT_CACHE[0](src, dst)


def _dense_body(x_ref, c0_ref, c1_ref, w1_ref, as1_ref, ad1_ref, b1_ref,
                w4_ref, as4_ref, ad4_ref, b4_ref, out_ref):
    f32 = jnp.float32
    C = c0_ref[:] + c1_ref[:]
    rows = lax.broadcasted_iota(jnp.int32, (N, N), 0)
    cols = lax.broadcasted_iota(jnp.int32, (N, N), 1)
    C = C + jnp.where(rows == cols, 1.0, 0.0).astype(f32)
    mask = C > 0.0

    def gat(h, att_s, att_d, b):
        a_s = lax.dot_general(att_s, h, (((1,), (1,)), ((), ())),
                              preferred_element_type=f32)
        a_d = jnp.sum(h * att_d, axis=1, keepdims=True)
        alpha = a_d + a_s
        alpha = jnp.where(alpha > 0.0, alpha, 0.2 * alpha)
        am = jnp.max(jnp.where(mask, alpha, -1e30), axis=1, keepdims=True)
        e = jnp.where(mask, C * jnp.exp(alpha - am), 0.0)
        denom = jnp.sum(e, axis=1, keepdims=True)
        coef = e / denom
        return jnp.dot(coef, h, preferred_element_type=f32) + b

    h1 = jnp.dot(x_ref[:], w1_ref[:], preferred_element_type=f32)
    h = jnp.maximum(gat(h1, as1_ref[:], ad1_ref[:], b1_ref[:]), 0.0)
    h2 = jnp.dot(h, w4_ref[:], preferred_element_type=f32)
    g = gat(h2, as4_ref[:], ad4_ref[:], b4_ref[:])
    out_ref[:] = jnp.where(g > 0.0, g, 0.01 * g)


def _fc_body(hr_ref, wfc_ref, bfc_ref, out_ref):
    o = jnp.dot(hr_ref[:], wfc_ref[:],
                preferred_element_type=jnp.float32) + bfc_ref[:]
    out_ref[:] = 1.0 / (1.0 + jnp.exp(-o))


def kernel(x_s, x_t, edge_index, edge_attr, batch, W1, att_src1, att_dst1,
           b1, W4, att_src4, att_dst4, b4, Wfc, bfc):
    x = jnp.concatenate([x_s, x_t], axis=0)
    x = jnp.pad(x, ((0, N - x.shape[0]), (0, 0)))

    padn = E_PAD - E
    src = jnp.concatenate(
        [edge_index[0], jnp.zeros((padn,), edge_index.dtype)])
    dst = jnp.concatenate(
        [edge_index[1], jnp.full((padn,), N, edge_index.dtype)])

    tbl = _sc_count(src, dst)
    c0 = tbl[:TBL].reshape(N, N)
    c1 = tbl[TBL_PAD:TBL_PAD + TBL].reshape(N, N)

    g = pl.pallas_call(
        _dense_body,
        out_shape=jax.ShapeDtypeStruct((N, BS), jnp.float32),
    )(x, c0, c1, W1,
      att_src1.reshape(1, HID), att_dst1.reshape(1, HID), b1.reshape(1, HID),
      W4, att_src4.reshape(1, BS), att_dst4.reshape(1, BS), b4.reshape(1, BS))

    hr = g.reshape(BS, N)
    out = pl.pallas_call(
        _fc_body,
        out_shape=jax.ShapeDtypeStruct((BS, 1), jnp.float32),
    )(hr, Wfc, bfc.reshape(1, 1))
    return out.reshape(1, BS)
